# Initial kernel scaffold; baseline (speedup 1.0000x reference)
import jax
import jax.numpy as jnp
from jax.experimental import pallas as pl


def kernel(x, table, W, b):
    raise NotImplementedError("write your pallas kernel here")


# speedup vs baseline: 7.7998x; 7.7998x over previous
import functools

import jax
import jax.numpy as jnp
from jax import lax
from jax.experimental import pallas as pl
from jax.experimental.pallas import tpu as pltpu
from jax.experimental.pallas import tpu_sc as plsc

NC, NS, L = 2, 16, 16
NW = NC * NS
B, SEQ, EMB = 16384, 200, 16
CHUNK = 16
GPC = CHUNK * SEQ
GB = 128
NGB = GPC // GB
CHUNKS_PER_W = B // (NW * CHUNK)
ROUND_MAGIC = 12582912.0


@functools.partial(
    pl.kernel,
    out_type=jax.ShapeDtypeStruct((B,), jnp.float32),
    mesh=plsc.VectorSubcoreMesh(
        core_axis_name="c", subcore_axis_name="s",
        num_cores=NC, num_subcores=NS),
    compiler_params=pltpu.CompilerParams(
        needs_layout_passes=False, use_tc_tiling_on_sc=False),
    scratch_types=[
        pltpu.VMEM((NGB, GB), jnp.int32),
        pltpu.VMEM((GPC, EMB), jnp.float32),
        pltpu.VMEM((2, L), jnp.float32),
        pltpu.VMEM((EMB, CHUNK), jnp.float32),
        pltpu.VMEM((L,), jnp.float32),
        pltpu.SemaphoreType.DMA,
    ],
)
def _sc_forward(xr_hbm, table_hbm, wb_hbm, out_hbm,
                idx_v, rows_v, wb_v, accT_v, out_v, sem):
    wid = lax.axis_index("s") * NC + lax.axis_index("c")
    pltpu.sync_copy(wb_hbm, wb_v)
    wv = wb_v[0, :]
    bv = wb_v[1, :]
    lane = lax.broadcasted_iota(jnp.int32, (L,), 0)

    def chunk_body(g, carry):
        ci = wid * CHUNKS_PER_W + g
        pltpu.sync_copy(xr_hbm.at[ci], idx_v)
        cps = [
            pltpu.async_copy(
                table_hbm.at[idx_v.at[j]],
                rows_v.at[pl.ds(j * GB, GB)],
                sem,
            )
            for j in range(NGB)
        ]
        for cp in cps:
            cp.wait()

        def row_body(c, carry_):
            base = c * SEQ

            def acc_body(l, acc):
                return acc + rows_v[base + l, :]

            acc = lax.fori_loop(0, SEQ, acc_body,
                                jnp.zeros((L,), jnp.float32), unroll=8)
            plsc.store_scatter(accT_v, [lane, jnp.full((L,), c, jnp.int32)],
                               acc * wv)
            return carry_

        lax.fori_loop(0, CHUNK, row_body, 0)
        zvec = accT_v[0, :]
        for e in range(1, EMB):
            zvec = zvec + accT_v[e, :]
        sig = 1.0 / (1.0 + jnp.exp(-(zvec + bv)))
        y = sig * 1e4
        r = (y + ROUND_MAGIC) - ROUND_MAGIC
        out_v[...] = r * 1e-4
        pltpu.sync_copy(out_v, out_hbm.at[pl.ds(ci * CHUNK, CHUNK)])
        return carry

    lax.fori_loop(0, CHUNKS_PER_W, chunk_body, 0)


def kernel(x, table, W, b):
    xr = x.astype(jnp.int32).reshape(B // CHUNK, NGB, GB)
    wv = (W.astype(jnp.float32).reshape(EMB) / SEQ)
    bv = jnp.broadcast_to(b.astype(jnp.float32), (L,))
    wb = jnp.stack([wv, bv])
    out = _sc_forward(xr, table, wb)
    return out.reshape(B, 1)


# speedup vs baseline: 9.7721x; 1.2529x over previous
import functools

import jax
import jax.numpy as jnp
from jax import lax
from jax.experimental import pallas as pl
from jax.experimental.pallas import tpu as pltpu
from jax.experimental.pallas import tpu_sc as plsc

NC, NS, L = 2, 16, 16
NW = NC * NS
B, SEQ, EMB = 16384, 200, 16
CHUNK = 16
GPC = CHUNK * SEQ
GB = 128
NGB = GPC // GB
CHUNKS_PER_W = B // (NW * CHUNK)
ROUND_MAGIC = 12582912.0


@functools.partial(
    pl.kernel,
    out_type=jax.ShapeDtypeStruct((B,), jnp.float32),
    mesh=plsc.VectorSubcoreMesh(
        core_axis_name="c", subcore_axis_name="s",
        num_cores=NC, num_subcores=NS),
    compiler_params=pltpu.CompilerParams(
        needs_layout_passes=False, use_tc_tiling_on_sc=False),
    scratch_types=[
        pltpu.VMEM((NGB, GB), jnp.int32),
        pltpu.VMEM((NGB, GB), jnp.int32),
        pltpu.VMEM((GPC, EMB), jnp.float32),
        pltpu.VMEM((GPC, EMB), jnp.float32),
        pltpu.VMEM((2, L), jnp.float32),
        pltpu.VMEM((EMB, CHUNK), jnp.float32),
        pltpu.VMEM((L,), jnp.float32),
        pltpu.SemaphoreType.DMA,
        pltpu.SemaphoreType.DMA,
    ],
)
def _sc_forward(xr_hbm, table_hbm, wb_hbm, out_hbm,
                idx0_v, idx1_v, rows0_v, rows1_v, wb_v, accT_v, out_v,
                sem0, sem1):
    wid = lax.axis_index("s") * NC + lax.axis_index("c")
    pltpu.sync_copy(wb_hbm, wb_v)
    wv = wb_v[0, :]
    bv = wb_v[1, :]
    lane = lax.broadcasted_iota(jnp.int32, (L,), 0)
    zero = jnp.zeros((L,), jnp.float32)

    def fire(ci, idx_v, rows_v, sem):
        pltpu.sync_copy(xr_hbm.at[ci], idx_v)
        for j in range(NGB):
            pltpu.async_copy(
                table_hbm.at[idx_v.at[j]],
                rows_v.at[pl.ds(j * GB, GB)],
                sem,
            )

    def drain(idx_v, rows_v, sem):
        for j in range(NGB):
            pltpu.make_async_copy(
                table_hbm.at[idx_v.at[j]],
                rows_v.at[pl.ds(j * GB, GB)],
                sem,
            ).wait()

    def reduce_store(rows_v, ci):
        def row_body(c, carry_):
            base = c * SEQ

            def acc_body(i, accs):
                b2 = base + i * 4
                a0, a1, a2, a3 = accs
                return (a0 + rows_v[b2, :], a1 + rows_v[b2 + 1, :],
                        a2 + rows_v[b2 + 2, :], a3 + rows_v[b2 + 3, :])

            a0, a1, a2, a3 = lax.fori_loop(
                0, SEQ // 4, acc_body, (zero, zero, zero, zero), unroll=5)
            acc = (a0 + a1) + (a2 + a3)
            plsc.store_scatter(accT_v, [lane, jnp.full((L,), c, jnp.int32)],
                               acc * wv)
            return carry_

        lax.fori_loop(0, CHUNK, row_body, 0)
        zvec = accT_v[0, :]
        for e in range(1, EMB):
            zvec = zvec + accT_v[e, :]
        sig = 1.0 / (1.0 + jnp.exp(-(zvec + bv)))
        y = sig * 1e4
        r = (y + ROUND_MAGIC) - ROUND_MAGIC
        out_v[...] = r * 1e-4
        pltpu.sync_copy(out_v, out_hbm.at[pl.ds(ci * CHUNK, CHUNK)])

    base_ci = wid * CHUNKS_PER_W
    fire(base_ci, idx0_v, rows0_v, sem0)

    def body2(k, carry):
        g0 = base_ci + 2 * k
        fire(g0 + 1, idx1_v, rows1_v, sem1)
        drain(idx0_v, rows0_v, sem0)
        reduce_store(rows0_v, g0)

        @pl.when(2 * k + 2 < CHUNKS_PER_W)
        def _():
            fire(g0 + 2, idx0_v, rows0_v, sem0)

        drain(idx1_v, rows1_v, sem1)
        reduce_store(rows1_v, g0 + 1)
        return carry

    lax.fori_loop(0, CHUNKS_PER_W // 2, body2, 0)


def kernel(x, table, W, b):
    xr = x.astype(jnp.int32).reshape(B // CHUNK, NGB, GB)
    wv = (W.astype(jnp.float32).reshape(EMB) / SEQ)
    bv = jnp.broadcast_to(b.astype(jnp.float32), (L,))
    wb = jnp.stack([wv, bv])
    out = _sc_forward(xr, table, wb)
    return out.reshape(B, 1)


# speedup vs baseline: 49.7172x; 5.0877x over previous
import functools

import jax
import jax.numpy as jnp
from jax import lax
from jax.experimental import pallas as pl
from jax.experimental.pallas import tpu as pltpu
from jax.experimental.pallas import tpu_sc as plsc

NC, NS, L = 2, 16, 16
NW = NC * NS
B, SEQ, EMB = 16384, 200, 16
VOCAB = 1000000
TW_N = 1 << 20
CHUNK = 16
GPC = CHUNK * SEQ
GB = 128
NGB = GPC // GB
CHUNKS_PER_W = B // (NW * CHUNK)
SPMEM_SLICE = TW_N // NS
TC_GRID = 8
TC_BLK = TW_N // TC_GRID
ROUND_MAGIC = 12582912.0


def _tw_body(wb_sref, tT_ref, out_ref):
    acc = tT_ref[0, :] * wb_sref[0, 0]
    for e in range(1, EMB):
        acc = acc + tT_ref[e, :] * wb_sref[0, e]
    out_ref[...] = acc


_tw_call = pl.pallas_call(
    _tw_body,
    grid=(TC_GRID,),
    in_specs=[
        pl.BlockSpec(memory_space=pltpu.SMEM),
        pl.BlockSpec((EMB, TC_BLK), lambda g: (0, g)),
    ],
    out_specs=pl.BlockSpec((TC_BLK,), lambda g: (g,)),
    out_shape=jax.ShapeDtypeStruct((TW_N,), jnp.float32),
)


@functools.partial(
    pl.kernel,
    out_type=jax.ShapeDtypeStruct((B,), jnp.float32),
    mesh=plsc.VectorSubcoreMesh(
        core_axis_name="c", subcore_axis_name="s",
        num_cores=NC, num_subcores=NS),
    compiler_params=pltpu.CompilerParams(
        needs_layout_passes=False, use_tc_tiling_on_sc=False),
    scratch_types=[
        pltpu.VMEM((NGB, GB), jnp.int32),
        pltpu.VMEM((NGB, GB), jnp.int32),
        pltpu.VMEM((GPC,), jnp.float32),
        pltpu.VMEM((GPC,), jnp.float32),
        pltpu.VMEM((2, L), jnp.float32),
        pltpu.VMEM((L,), jnp.float32),
        pltpu.VMEM_SHARED((TW_N,), jnp.float32),
        pltpu.SemaphoreType.DMA,
        pltpu.SemaphoreType.DMA,
    ],
)
def _sc_forward(xr_hbm, tw_hbm, wb_hbm, out_hbm,
                idx0_v, idx1_v, vals0_v, vals1_v, wb_v, out_v, tw_sh,
                sem0, sem1):
    cid = lax.axis_index("c")
    sid = lax.axis_index("s")
    wid = sid * NC + cid
    pltpu.sync_copy(tw_hbm.at[pl.ds(sid * SPMEM_SLICE, SPMEM_SLICE)],
                    tw_sh.at[pl.ds(sid * SPMEM_SLICE, SPMEM_SLICE)])
    pltpu.sync_copy(wb_hbm, wb_v)
    plsc.subcore_barrier()

    bv = wb_v[1, :]
    lane = lax.broadcasted_iota(jnp.int32, (L,), 0)
    base_idx = lane * SEQ
    zero = jnp.zeros((L,), jnp.float32)

    def fire(ci, idx_v, vals_v, sem):
        pltpu.sync_copy(xr_hbm.at[ci], idx_v)
        for j in range(NGB):
            pltpu.async_copy(
                tw_sh.at[idx_v.at[j]],
                vals_v.at[pl.ds(j * GB, GB)],
                sem,
            )

    def drain(idx_v, vals_v, sem):
        for j in range(NGB):
            pltpu.make_async_copy(
                tw_sh.at[idx_v.at[j]],
                vals_v.at[pl.ds(j * GB, GB)],
                sem,
            ).wait()

    def reduce_store(vals_v, ci):
        def acc_body(i, accs):
            l = i * 4
            a0, a1, a2, a3 = accs
            return (a0 + plsc.load_gather(vals_v, [base_idx + l]),
                    a1 + plsc.load_gather(vals_v, [base_idx + (l + 1)]),
                    a2 + plsc.load_gather(vals_v, [base_idx + (l + 2)]),
                    a3 + plsc.load_gather(vals_v, [base_idx + (l + 3)]))

        a0, a1, a2, a3 = lax.fori_loop(
            0, SEQ // 4, acc_body, (zero, zero, zero, zero), unroll=5)
        z = (a0 + a1) + (a2 + a3) + bv
        sig = 1.0 / (1.0 + jnp.exp(-z))
        y = sig * 1e4
        r = (y + ROUND_MAGIC) - ROUND_MAGIC
        out_v[...] = r * 1e-4
        pltpu.sync_copy(out_v, out_hbm.at[pl.ds(ci * CHUNK, CHUNK)])

    base_ci = wid * CHUNKS_PER_W
    fire(base_ci, idx0_v, vals0_v, sem0)

    def body2(k, carry):
        g0 = base_ci + 2 * k
        fire(g0 + 1, idx1_v, vals1_v, sem1)
        drain(idx0_v, vals0_v, sem0)
        reduce_store(vals0_v, g0)

        @pl.when(2 * k + 2 < CHUNKS_PER_W)
        def _():
            fire(g0 + 2, idx0_v, vals0_v, sem0)

        drain(idx1_v, vals1_v, sem1)
        reduce_store(vals1_v, g0 + 1)
        return carry

    lax.fori_loop(0, CHUNKS_PER_W // 2, body2, 0)


def kernel(x, table, W, b):
    tableT = jnp.transpose(table.astype(jnp.float32))
    wv = (W.astype(jnp.float32).reshape(EMB) / SEQ)
    bv = jnp.broadcast_to(b.astype(jnp.float32), (L,))
    wb = jnp.stack([wv, bv])
    tw = _tw_call(wb, tableT)
    xr = x.astype(jnp.int32).reshape(B // CHUNK, NGB, GB)
    out = _sc_forward(xr, tw, wb)
    return out.reshape(B, 1)


# speedup vs baseline: 52.9303x; 1.0646x over previous
import functools

import jax
import jax.numpy as jnp
from jax import lax
from jax.experimental import pallas as pl
from jax.experimental.pallas import tpu as pltpu
from jax.experimental.pallas import tpu_sc as plsc

NC, NS, L = 2, 16, 16
NW = NC * NS
B, SEQ, EMB = 16384, 200, 16
VOCAB = 1000000
TW_N = 1 << 20
CHUNK = 16
GPC = CHUNK * SEQ
GB = 128
NGB = GPC // GB
CHUNKS_PER_W = B // (NW * CHUNK)
SPMEM_SLICE = TW_N // NS
TC_GRID = 8
TC_BLK = TW_N // TC_GRID
ROUND_MAGIC = 12582912.0


def _tw_body(wb_sref, tT_ref, out_ref):
    acc = tT_ref[0, :] * wb_sref[0, 0]
    for e in range(1, EMB):
        acc = acc + tT_ref[e, :] * wb_sref[0, e]
    out_ref[...] = acc


_tw_call = pl.pallas_call(
    _tw_body,
    grid=(TC_GRID,),
    in_specs=[
        pl.BlockSpec(memory_space=pltpu.SMEM),
        pl.BlockSpec((EMB, TC_BLK), lambda g: (0, g)),
    ],
    out_specs=pl.BlockSpec((TC_BLK,), lambda g: (g,)),
    out_shape=jax.ShapeDtypeStruct((TW_N,), jnp.float32),
)


@functools.partial(
    pl.kernel,
    out_type=jax.ShapeDtypeStruct((B,), jnp.float32),
    mesh=plsc.VectorSubcoreMesh(
        core_axis_name="c", subcore_axis_name="s",
        num_cores=NC, num_subcores=NS),
    compiler_params=pltpu.CompilerParams(
        needs_layout_passes=False, use_tc_tiling_on_sc=False),
    scratch_types=[
        pltpu.VMEM((GPC,), jnp.int32),
        pltpu.VMEM((GPC,), jnp.int32),
        pltpu.VMEM((GPC,), jnp.float32),
        pltpu.VMEM((GPC,), jnp.float32),
        pltpu.VMEM((2, L), jnp.float32),
        pltpu.VMEM((L,), jnp.float32),
        pltpu.VMEM_SHARED((TW_N,), jnp.float32),
        pltpu.SemaphoreType.DMA,
        pltpu.SemaphoreType.DMA,
        pltpu.SemaphoreType.DMA,
        pltpu.SemaphoreType.DMA,
    ],
)
def _sc_forward(xr_hbm, tw_hbm, wb_hbm, out_hbm,
                idx0_v, idx1_v, vals0_v, vals1_v, wb_v, out_v, tw_sh,
                sem0, sem1, semi0, semi1):
    cid = lax.axis_index("c")
    sid = lax.axis_index("s")
    wid = sid * NC + cid
    pltpu.sync_copy(tw_hbm.at[pl.ds(sid * SPMEM_SLICE, SPMEM_SLICE)],
                    tw_sh.at[pl.ds(sid * SPMEM_SLICE, SPMEM_SLICE)])
    pltpu.sync_copy(wb_hbm, wb_v)
    plsc.subcore_barrier()

    bv = wb_v[1, :]
    lane = lax.broadcasted_iota(jnp.int32, (L,), 0)
    base_idx = lane * SEQ
    zero = jnp.zeros((L,), jnp.float32)

    def start_idx(ci, idx_v, semi):
        pltpu.async_copy(xr_hbm.at[ci], idx_v, semi)

    def wait_idx(ci, idx_v, semi):
        pltpu.make_async_copy(xr_hbm.at[ci], idx_v, semi).wait()

    def fire(idx_v, vals_v, sem):
        pltpu.async_copy(tw_sh.at[idx_v], vals_v, sem)

    def drain(idx_v, vals_v, sem):
        pltpu.make_async_copy(tw_sh.at[idx_v], vals_v, sem).wait()

    def reduce_store(vals_v, ci):
        def acc_body(i, accs):
            l = i * 4
            a0, a1, a2, a3 = accs
            return (a0 + plsc.load_gather(vals_v, [base_idx + l]),
                    a1 + plsc.load_gather(vals_v, [base_idx + (l + 1)]),
                    a2 + plsc.load_gather(vals_v, [base_idx + (l + 2)]),
                    a3 + plsc.load_gather(vals_v, [base_idx + (l + 3)]))

        a0, a1, a2, a3 = lax.fori_loop(
            0, SEQ // 4, acc_body, (zero, zero, zero, zero), unroll=5)
        z = (a0 + a1) + (a2 + a3) + bv
        sig = 1.0 / (1.0 + jnp.exp(-z))
        y = sig * 1e4
        r = (y + ROUND_MAGIC) - ROUND_MAGIC
        out_v[...] = r * 1e-4
        pltpu.sync_copy(out_v, out_hbm.at[pl.ds(ci * CHUNK, CHUNK)])

    base_ci = wid * CHUNKS_PER_W
    start_idx(base_ci, idx0_v, semi0)
    wait_idx(base_ci, idx0_v, semi0)
    fire(idx0_v, vals0_v, sem0)
    start_idx(base_ci + 1, idx1_v, semi1)

    def body2(k, carry):
        g0 = base_ci + 2 * k
        wait_idx(g0 + 1, idx1_v, semi1)
        fire(idx1_v, vals1_v, sem1)

        @pl.when(2 * k + 2 < CHUNKS_PER_W)
        def _():
            start_idx(g0 + 2, idx0_v, semi0)

        drain(idx0_v, vals0_v, sem0)
        reduce_store(vals0_v, g0)

        @pl.when(2 * k + 2 < CHUNKS_PER_W)
        def _():
            wait_idx(g0 + 2, idx0_v, semi0)
            fire(idx0_v, vals0_v, sem0)

        @pl.when(2 * k + 3 < CHUNKS_PER_W)
        def _():
            start_idx(g0 + 3, idx1_v, semi1)

        drain(idx1_v, vals1_v, sem1)
        reduce_store(vals1_v, g0 + 1)
        return carry

    lax.fori_loop(0, CHUNKS_PER_W // 2, body2, 0)


def kernel(x, table, W, b):
    tableT = jnp.transpose(table.astype(jnp.float32))
    wv = (W.astype(jnp.float32).reshape(EMB) / SEQ)
    bv = jnp.broadcast_to(b.astype(jnp.float32), (L,))
    wb = jnp.stack([wv, bv])
    tw = _tw_call(wb, tableT)
    xr = x.astype(jnp.int32).reshape(B // CHUNK, GPC)
    out = _sc_forward(xr, tw, wb)
    return out.reshape(B, 1)
